# baseline (speedup 1.0000x reference)
import dataclasses
import functools
import math

import jax
import jax.numpy as jnp
from jax import lax
from jax.experimental import pallas as pl
from jax.experimental.pallas import tpu as pltpu
from jax.experimental.pallas import tpu_sc as plsc

_EPS = 1e-6
_INV_PI = 1.0 / math.pi

_NUM_CORES = 2
_NUM_SUBCORES = 16
_LANES = 16
_NUM_WORKERS = _NUM_CORES * _NUM_SUBCORES


def _log_radius_tc(seg_t, res2d, h_blk):
    h, w, b = seg_t.shape
    n_h = h // h_blk

    def body(res_ref, seg_ref, out_ref, acc_ref):
        step = pl.program_id(0)
        part = jnp.sum(seg_ref[...], axis=0)
        part = jnp.sum(part, axis=0, keepdims=True)

        @pl.when(step == 0)
        def _():
            acc_ref[...] = jnp.zeros_like(acc_ref)

        acc_ref[...] += part

        @pl.when(step == n_h - 1)
        def _():
            area = acc_ref[...] * res_ref[...] * res_ref[...]
            radius = jnp.sqrt(area * _INV_PI)
            out_ref[...] = jnp.log(jnp.maximum(radius, _EPS))

    return pl.pallas_call(
        body,
        grid=(n_h,),
        in_specs=[
            pl.BlockSpec((1, b), lambda i: (0, 0)),
            pl.BlockSpec((h_blk, w, b), lambda i: (i, 0, 0)),
        ],
        out_specs=pl.BlockSpec((1, b), lambda i: (0, 0)),
        out_shape=jax.ShapeDtypeStruct((1, b), jnp.float32),
        scratch_shapes=[pltpu.VMEM((1, b), jnp.float32)],
    )(res2d, seg_t)


def _finalize_sc(cat, logr, slopes_pad, icepts_pad):
    b = cat.shape[0]
    bpw = b // _NUM_WORKERS
    mesh = plsc.VectorSubcoreMesh(core_axis_name="c", subcore_axis_name="s")
    cp = pltpu.CompilerParams()
    if "needs_layout_passes" in pltpu.CompilerParams.__dataclass_fields__:
        cp = dataclasses.replace(cp, needs_layout_passes=False)

    @functools.partial(
        pl.kernel,
        mesh=mesh,
        compiler_params=cp,
        out_type=jax.ShapeDtypeStruct((b,), jnp.float32),
        scratch_types=[
            pltpu.VMEM((bpw,), jnp.int32),
            pltpu.VMEM((bpw,), jnp.float32),
            pltpu.VMEM((_LANES,), jnp.float32),
            pltpu.VMEM((_LANES,), jnp.float32),
            pltpu.VMEM((bpw,), jnp.float32),
        ],
    )
    def body(cat_hbm, lr_hbm, s_hbm, ic_hbm, out_hbm, idx_v, lr_v, s_v, ic_v, o_v):
        wid = lax.axis_index("s") * _NUM_CORES + lax.axis_index("c")
        base = wid * bpw
        pltpu.sync_copy(cat_hbm.at[pl.ds(base, bpw)], idx_v)
        pltpu.sync_copy(lr_hbm.at[pl.ds(base, bpw)], lr_v)
        pltpu.sync_copy(s_hbm, s_v)
        pltpu.sync_copy(ic_hbm, ic_v)
        for j in range(bpw // _LANES):
            sl = pl.ds(j * _LANES, _LANES)
            idx = idx_v[sl]
            s = plsc.load_gather(s_v, [idx])
            ic = plsc.load_gather(ic_v, [idx])
            o_v[sl] = jnp.exp(s * lr_v[sl] + ic)
        pltpu.sync_copy(o_v, out_hbm.at[pl.ds(base, bpw)])

    return body(cat, logr, slopes_pad, icepts_pad)


def kernel(category_id, segmentation, resolution, slopes, intercepts):
    seg = segmentation
    if seg.ndim == 4:
        seg = seg[:, 0]
    b, h, w = seg.shape
    seg_t = jnp.transpose(seg, (1, 2, 0))
    res2d = resolution.astype(jnp.float32).reshape(1, b)
    logr = _log_radius_tc(seg_t, res2d, h_blk=4)

    cat = category_id.astype(jnp.int32)
    pad = (-slopes.shape[0]) % _LANES
    slopes_pad = jnp.pad(slopes.astype(jnp.float32), (0, pad))
    icepts_pad = jnp.pad(intercepts.astype(jnp.float32), (0, pad))
    return _finalize_sc(cat, logr.reshape(b), slopes_pad, icepts_pad)


# speedup vs baseline: 1.1138x; 1.1138x over previous
import dataclasses
import functools
import math

import jax
import jax.numpy as jnp
from jax import lax
from jax.experimental import pallas as pl
from jax.experimental.pallas import tpu as pltpu
from jax.experimental.pallas import tpu_sc as plsc

_EPS = 1e-6
_INV_PI = 1.0 / math.pi

_NUM_CORES = 2
_NUM_SUBCORES = 16
_LANES = 16
_NUM_WORKERS = _NUM_CORES * _NUM_SUBCORES


def _log_radius_tc(seg_t, res2d, h_blk):
    h, w, b = seg_t.shape
    n_h = h // h_blk

    def body(res_ref, seg_ref, out_ref, acc_ref):
        step = pl.program_id(0)
        part = jnp.sum(seg_ref[...], axis=0)
        part = jnp.sum(part, axis=0, keepdims=True)

        @pl.when(step == 0)
        def _():
            acc_ref[...] = jnp.zeros_like(acc_ref)

        acc_ref[...] += part

        @pl.when(step == n_h - 1)
        def _():
            area = acc_ref[...] * res_ref[...] * res_ref[...]
            radius = jnp.sqrt(area * _INV_PI)
            out_ref[...] = jnp.log(jnp.maximum(radius, _EPS))

    return pl.pallas_call(
        body,
        grid=(n_h,),
        in_specs=[
            pl.BlockSpec((1, b), lambda i: (0, 0)),
            pl.BlockSpec((h_blk, w, b), lambda i: (i, 0, 0)),
        ],
        out_specs=pl.BlockSpec((1, b), lambda i: (0, 0)),
        out_shape=jax.ShapeDtypeStruct((1, b), jnp.float32),
        scratch_shapes=[pltpu.VMEM((1, b), jnp.float32)],
    )(res2d, seg_t)


def _finalize_sc(cat, logr, slopes_pad, icepts_pad):
    b = cat.shape[0]
    bpw = b // _NUM_WORKERS
    mesh = plsc.VectorSubcoreMesh(core_axis_name="c", subcore_axis_name="s")
    cp = pltpu.CompilerParams()
    if "needs_layout_passes" in pltpu.CompilerParams.__dataclass_fields__:
        cp = dataclasses.replace(cp, needs_layout_passes=False)

    @functools.partial(
        pl.kernel,
        mesh=mesh,
        compiler_params=cp,
        out_type=jax.ShapeDtypeStruct((b,), jnp.float32),
        scratch_types=[
            pltpu.VMEM((bpw,), jnp.int32),
            pltpu.VMEM((bpw,), jnp.float32),
            pltpu.VMEM((_LANES,), jnp.float32),
            pltpu.VMEM((_LANES,), jnp.float32),
            pltpu.VMEM((bpw,), jnp.float32),
        ],
    )
    def body(cat_hbm, lr_hbm, s_hbm, ic_hbm, out_hbm, idx_v, lr_v, s_v, ic_v, o_v):
        wid = lax.axis_index("s") * _NUM_CORES + lax.axis_index("c")
        base = wid * bpw
        pltpu.sync_copy(cat_hbm.at[pl.ds(base, bpw)], idx_v)
        pltpu.sync_copy(lr_hbm.at[pl.ds(base, bpw)], lr_v)
        pltpu.sync_copy(s_hbm, s_v)
        pltpu.sync_copy(ic_hbm, ic_v)
        for j in range(bpw // _LANES):
            sl = pl.ds(j * _LANES, _LANES)
            idx = idx_v[sl]
            s = plsc.load_gather(s_v, [idx])
            ic = plsc.load_gather(ic_v, [idx])
            o_v[sl] = jnp.exp(s * lr_v[sl] + ic)
        pltpu.sync_copy(o_v, out_hbm.at[pl.ds(base, bpw)])

    return body(cat, logr, slopes_pad, icepts_pad)


def kernel(category_id, segmentation, resolution, slopes, intercepts):
    seg = segmentation
    if seg.ndim == 4:
        seg = seg[:, 0]
    b, h, w = seg.shape
    seg_t = jnp.transpose(seg, (1, 2, 0))
    res2d = resolution.astype(jnp.float32).reshape(1, b)
    logr = _log_radius_tc(seg_t, res2d, h_blk=8)

    cat = category_id.astype(jnp.int32)
    pad = (-slopes.shape[0]) % _LANES
    slopes_pad = jnp.pad(slopes.astype(jnp.float32), (0, pad))
    icepts_pad = jnp.pad(intercepts.astype(jnp.float32), (0, pad))
    return _finalize_sc(cat, logr.reshape(b), slopes_pad, icepts_pad)


# speedup vs baseline: 1.1638x; 1.0449x over previous
import dataclasses
import functools
import math

import jax
import jax.numpy as jnp
from jax import lax
from jax.experimental import pallas as pl
from jax.experimental.pallas import tpu as pltpu
from jax.experimental.pallas import tpu_sc as plsc

_EPS = 1e-6
_INV_PI = 1.0 / math.pi

_NUM_CORES = 2
_NUM_SUBCORES = 16
_LANES = 16
_NUM_WORKERS = _NUM_CORES * _NUM_SUBCORES


def _log_radius_tc(seg_t, res2d, h_blk):
    h, w, b = seg_t.shape
    n_h = h // h_blk

    n_t = b // 128

    def body(res_ref, seg_ref, out_ref, acc_ref):
        step = pl.program_id(0)
        part = jnp.sum(seg_ref[...], axis=0)
        part = jnp.sum(part, axis=0, keepdims=True)

        @pl.when(step == 0)
        def _():
            acc_ref[...] = jnp.zeros_like(acc_ref)

        acc_ref[...] += part

        @pl.when(step == n_h - 1)
        def _():
            for t in range(n_t):
                pc = acc_ref[:, pl.ds(t * 128, 128)]
                rv = res_ref[pl.ds(t, 1), :]
                area = pc * rv * rv
                radius = jnp.sqrt(area * _INV_PI)
                out_ref[pl.ds(t, 1), :] = jnp.log(jnp.maximum(radius, _EPS))

    return pl.pallas_call(
        body,
        grid=(n_h,),
        in_specs=[
            pl.BlockSpec((n_t, 128), lambda i: (0, 0)),
            pl.BlockSpec((h_blk, w, b), lambda i: (i, 0, 0)),
        ],
        out_specs=pl.BlockSpec((n_t, 128), lambda i: (0, 0)),
        out_shape=jax.ShapeDtypeStruct((n_t, 128), jnp.float32),
        scratch_shapes=[pltpu.VMEM((1, b), jnp.float32)],
    )(res2d, seg_t)


def _finalize_sc(cat, logr, slopes, icepts):
    b = cat.shape[0]
    bpw = b // _NUM_WORKERS
    n_cat = slopes.shape[0]
    mesh = plsc.VectorSubcoreMesh(core_axis_name="c", subcore_axis_name="s")
    cp = pltpu.CompilerParams()
    if "needs_layout_passes" in pltpu.CompilerParams.__dataclass_fields__:
        cp = dataclasses.replace(cp, needs_layout_passes=False)

    @functools.partial(
        pl.kernel,
        mesh=mesh,
        compiler_params=cp,
        out_type=jax.ShapeDtypeStruct((b,), jnp.float32),
        scratch_types=[
            pltpu.VMEM((bpw,), jnp.int32),
            pltpu.VMEM((bpw,), jnp.float32),
            pltpu.VMEM((n_cat,), jnp.float32),
            pltpu.VMEM((n_cat,), jnp.float32),
            pltpu.VMEM((bpw,), jnp.float32),
            pltpu.SemaphoreType.DMA,
            pltpu.SemaphoreType.DMA,
            pltpu.SemaphoreType.DMA,
            pltpu.SemaphoreType.DMA,
        ],
    )
    def body(cat_hbm, lr_hbm, s_hbm, ic_hbm, out_hbm,
             idx_v, lr_v, s_v, ic_v, o_v, sem0, sem1, sem2, sem3):
        wid = lax.axis_index("s") * _NUM_CORES + lax.axis_index("c")
        base = wid * bpw
        c0 = pltpu.async_copy(cat_hbm.at[pl.ds(base, bpw)], idx_v, sem0)
        c1 = pltpu.async_copy(lr_hbm.at[pl.ds(base, bpw)], lr_v, sem1)
        c2 = pltpu.async_copy(s_hbm, s_v, sem2)
        c3 = pltpu.async_copy(ic_hbm, ic_v, sem3)
        c0.wait()
        c1.wait()
        c2.wait()
        c3.wait()
        for j in range(bpw // _LANES):
            sl = pl.ds(j * _LANES, _LANES)
            idx = idx_v[sl]
            s = plsc.load_gather(s_v, [idx])
            ic = plsc.load_gather(ic_v, [idx])
            o_v[sl] = jnp.exp(s * lr_v[sl] + ic)
        pltpu.sync_copy(o_v, out_hbm.at[pl.ds(base, bpw)])

    return body(cat, logr, slopes, icepts)


def kernel(category_id, segmentation, resolution, slopes, intercepts):
    seg = segmentation
    if seg.ndim == 4:
        seg = seg[:, 0]
    b, h, w = seg.shape
    seg_t = jnp.transpose(seg, (1, 2, 0))
    res2d = resolution.astype(jnp.float32).reshape(b // 128, 128)
    logr = _log_radius_tc(seg_t, res2d, h_blk=8)

    cat = category_id.astype(jnp.int32)
    return _finalize_sc(cat, logr.reshape(b), slopes.astype(jnp.float32),
                        intercepts.astype(jnp.float32))
